# baseline (speedup 1.0000x reference)
import functools

import jax
import jax.numpy as jnp
from jax import lax
from jax.experimental import pallas as pl
from jax.experimental.pallas import tpu as pltpu
from jax.experimental.pallas import tpu_sc as plsc

N_ENTITY = 100000
N_NEIGHBOR = 16
BATCH = 16384

NC = 2
NS = 16
NW = NC * NS
SPW = BATCH // NW
WIDE = N_NEIGHBOR * N_NEIGHBOR


def _rf_body(x_hbm, ent_hbm, rel_hbm,
             out1, out2, out3, out4,
             idx0_v, ent1_v, rel1_v, ent1t_v, ent2_v, rel2_v,
             sem_h1e, sem_h1r, sem_e0, sem_e1, sem_r0, sem_r1):
    wid = lax.axis_index("s") * NC + lax.axis_index("c")
    base = wid * SPW

    pltpu.sync_copy(x_hbm.at[pl.ds(base, SPW)], idx0_v)

    cp_e1 = pltpu.async_copy(ent_hbm.at[idx0_v], ent1_v, sem_h1e)
    cp_r1 = pltpu.async_copy(rel_hbm.at[idx0_v], rel1_v, sem_h1r)
    cp_e1.wait()

    rows16 = lax.iota(jnp.int32, 16)
    for k in range(N_NEIGHBOR):
        col = jnp.full((16,), k, jnp.int32)
        for g in range(SPW // 16):
            v = plsc.load_gather(ent1_v, [rows16 + (g * 16), col])
            ent1t_v[k, pl.ds(g * 16, 16)] = v

    sem_e = (sem_e0, sem_e1)
    sem_r = (sem_r0, sem_r1)
    cp_e = [None, None]
    cp_r = [None, None]

    for k in range(N_NEIGHBOR + 1):
        if k < N_NEIGHBOR:
            b = k % 2
            idx_k = ent1t_v.at[k]
            cp_e[b] = pltpu.async_copy(ent_hbm.at[idx_k], ent2_v.at[b], sem_e[b])
            cp_r[b] = pltpu.async_copy(rel_hbm.at[idx_k], rel2_v.at[b], sem_r[b])
        if k == 0:
            pltpu.sync_copy(ent1_v, out1.at[pl.ds(base, SPW)])
            cp_r1.wait()
            pltpu.sync_copy(rel1_v, out3.at[pl.ds(base, SPW)])
        else:
            pb = (k - 1) % 2
            cols = pl.ds((k - 1) * N_NEIGHBOR, N_NEIGHBOR)
            cp_e[pb].wait()
            pltpu.sync_copy(ent2_v.at[pb], out2.at[pl.ds(base, SPW), cols])
            cp_r[pb].wait()
            pltpu.sync_copy(rel2_v.at[pb], out4.at[pl.ds(base, SPW), cols])


@jax.jit
def kernel(x, adj_entity, adj_relation):
    x_flat = x.reshape(BATCH).astype(jnp.int32)
    ent = adj_entity.astype(jnp.int32)
    rel = adj_relation.astype(jnp.int32)

    i32 = jnp.int32
    run = pl.kernel(
        _rf_body,
        out_type=(
            jax.ShapeDtypeStruct((BATCH, N_NEIGHBOR), i32),
            jax.ShapeDtypeStruct((BATCH, WIDE), i32),
            jax.ShapeDtypeStruct((BATCH, N_NEIGHBOR), i32),
            jax.ShapeDtypeStruct((BATCH, WIDE), i32),
        ),
        mesh=plsc.VectorSubcoreMesh(core_axis_name="c", subcore_axis_name="s"),
        compiler_params=pltpu.CompilerParams(use_tc_tiling_on_sc=False, needs_layout_passes=False),
        scratch_types=[
            pltpu.VMEM((SPW,), i32),
            pltpu.VMEM((SPW, N_NEIGHBOR), i32),
            pltpu.VMEM((SPW, N_NEIGHBOR), i32),
            pltpu.VMEM((N_NEIGHBOR, SPW), i32),
            pltpu.VMEM((2, SPW, N_NEIGHBOR), i32),
            pltpu.VMEM((2, SPW, N_NEIGHBOR), i32),
            pltpu.SemaphoreType.DMA,
            pltpu.SemaphoreType.DMA,
            pltpu.SemaphoreType.DMA,
            pltpu.SemaphoreType.DMA,
            pltpu.SemaphoreType.DMA,
            pltpu.SemaphoreType.DMA,
        ],
    )
    ent1, ent2, rel1, rel2 = run(x_flat, ent, rel)
    return (x, ent1, ent2, rel1, rel2)


# speedup vs baseline: 1.0669x; 1.0669x over previous
import functools

import jax
import jax.numpy as jnp
from jax import lax
from jax.experimental import pallas as pl
from jax.experimental.pallas import tpu as pltpu
from jax.experimental.pallas import tpu_sc as plsc

N_ENTITY = 100000
N_NEIGHBOR = 16
BATCH = 16384

NC = 2
NS = 16
NW = NC * NS
SPW = BATCH // NW
WIDE = N_NEIGHBOR * N_NEIGHBOR


def _rf_body(x_hbm, ent_hbm, rel_hbm,
             out1t, out2, out3, out4,
             idx0_v, ent1_v, rel1_v, ent1t_v, ent2_v, rel2_v,
             sem_h1e, sem_h1r, sem_e0, sem_e1, sem_r0, sem_r1):
    wid = lax.axis_index("s") * NC + lax.axis_index("c")
    base = wid * SPW

    pltpu.sync_copy(x_hbm.at[pl.ds(base, SPW)], idx0_v)

    cp_e1 = pltpu.async_copy(ent_hbm.at[idx0_v], ent1_v, sem_h1e)
    cp_r1 = pltpu.async_copy(rel_hbm.at[idx0_v], rel1_v, sem_h1r)
    cp_e1.wait()

    rows16 = lax.iota(jnp.int32, 16)

    def transpose_col(k):
        col = jnp.full((16,), k, jnp.int32)
        for g in range(SPW // 16):
            v = plsc.load_gather(ent1_v, [rows16 + (g * 16), col])
            ent1t_v[k, pl.ds(g * 16, 16)] = v

    transpose_col(0)

    sem_e = (sem_e0, sem_e1)
    sem_r = (sem_r0, sem_r1)
    cp_e = [None, None]
    cp_r = [None, None]

    for k in range(N_NEIGHBOR + 1):
        if k < N_NEIGHBOR:
            b = k % 2
            idx_k = ent1t_v.at[k]
            cp_e[b] = pltpu.async_copy(ent_hbm.at[idx_k], ent2_v.at[b], sem_e[b])
            cp_r[b] = pltpu.async_copy(rel_hbm.at[idx_k], rel2_v.at[b], sem_r[b])
        if k + 1 < N_NEIGHBOR:
            transpose_col(k + 1)
        if k == N_NEIGHBOR - 1:
            pltpu.sync_copy(ent1t_v, out1t.at[:, pl.ds(base, SPW)])
        if k == 0:
            cp_r1.wait()
            pltpu.sync_copy(rel1_v, out3.at[pl.ds(base, SPW)])
        else:
            pb = (k - 1) % 2
            cols = pl.ds((k - 1) * N_NEIGHBOR, N_NEIGHBOR)
            cp_e[pb].wait()
            pltpu.sync_copy(ent2_v.at[pb], out2.at[pl.ds(base, SPW), cols])
            cp_r[pb].wait()
            pltpu.sync_copy(rel2_v.at[pb], out4.at[pl.ds(base, SPW), cols])


@jax.jit
def kernel(x, adj_entity, adj_relation):
    x_flat = x.reshape(BATCH).astype(jnp.int32)
    ent_flat, rel_flat = lax.optimization_barrier(
        (adj_entity.astype(jnp.int32).reshape(-1),
         adj_relation.astype(jnp.int32).reshape(-1)))
    ent = ent_flat.reshape(N_ENTITY, N_NEIGHBOR)
    rel = rel_flat.reshape(N_ENTITY, N_NEIGHBOR)

    i32 = jnp.int32
    run = pl.kernel(
        _rf_body,
        out_type=(
            jax.ShapeDtypeStruct((N_NEIGHBOR, BATCH), i32),
            jax.ShapeDtypeStruct((BATCH, WIDE), i32),
            jax.ShapeDtypeStruct((BATCH, N_NEIGHBOR), i32),
            jax.ShapeDtypeStruct((BATCH, WIDE), i32),
        ),
        mesh=plsc.VectorSubcoreMesh(core_axis_name="c", subcore_axis_name="s"),
        compiler_params=pltpu.CompilerParams(
            use_tc_tiling_on_sc=False, needs_layout_passes=False),
        scratch_types=[
            pltpu.VMEM((SPW,), i32),
            pltpu.VMEM((SPW, N_NEIGHBOR), i32),
            pltpu.VMEM((SPW, N_NEIGHBOR), i32),
            pltpu.VMEM((N_NEIGHBOR, SPW), i32),
            pltpu.VMEM((2, SPW, N_NEIGHBOR), i32),
            pltpu.VMEM((2, SPW, N_NEIGHBOR), i32),
            pltpu.SemaphoreType.DMA,
            pltpu.SemaphoreType.DMA,
            pltpu.SemaphoreType.DMA,
            pltpu.SemaphoreType.DMA,
            pltpu.SemaphoreType.DMA,
            pltpu.SemaphoreType.DMA,
        ],
    )
    ent1t, ent2, rel1, rel2 = run(x_flat, ent, rel)
    return (x, ent1t.T, ent2, rel1, rel2)
